# Initial kernel scaffold; baseline (speedup 1.0000x reference)
import jax
import jax.numpy as jnp
from jax.experimental import pallas as pl


def kernel(x, edge_index, edge_weight, Wenc, benc, Wb, Wdec, beta_raw, gamma_raw):
    raise NotImplementedError("write your pallas kernel here")


# speedup vs baseline: 1.3172x; 1.3172x over previous
import functools

import jax
import jax.numpy as jnp
from jax import lax
from jax.experimental import pallas as pl
from jax.experimental.pallas import tpu as pltpu
from jax.experimental.pallas import tpu_sc as plsc

N_NODES = 10000
N_PAD = 10240
N_EDGES = 320000
IN_CH = 128
HID_CH = 256
OUT_CH = 128
MAX_ITER = 8

NC = 2
NS = 16
CBLK = 64
NBLK_PER_CORE = 2

K = 128
EPAD = N_EDGES + K
ROWS_PT = N_PAD // NS
RCH = 64
NRCH = ROWS_PT // RCH

MB = 1024


def _binj_body(x_ref, wenc_ref, benc_ref, wb_ref, o_ref):
    h = lax.dot_general(x_ref[...], wenc_ref[...], (((1,), (1,)), ((), ())),
                        preferred_element_type=jnp.float32)
    h = h + benc_ref[...]
    o_ref[...] = lax.dot_general(h, wb_ref[...], (((1,), (1,)), ((), ())),
                                 preferred_element_type=jnp.float32)


def _compute_binj(xp, wenc, benc2, wb):
    return pl.pallas_call(
        _binj_body,
        grid=(N_PAD // MB,),
        in_specs=[
            pl.BlockSpec((MB, IN_CH), lambda i: (i, 0)),
            pl.BlockSpec((HID_CH, IN_CH), lambda i: (0, 0)),
            pl.BlockSpec((1, HID_CH), lambda i: (0, 0)),
            pl.BlockSpec((HID_CH, HID_CH), lambda i: (0, 0)),
        ],
        out_specs=pl.BlockSpec((MB, HID_CH), lambda i: (i, 0)),
        out_shape=jax.ShapeDtypeStruct((N_PAD, HID_CH), jnp.float32),
    )(xp, wenc, benc2, wb)


def _dec_body(u_ref, wd_ref, o_ref):
    o_ref[...] = lax.dot_general(jnp.maximum(u_ref[...], 0.0), wd_ref[...],
                                 (((1,), (1,)), ((), ())),
                                 preferred_element_type=jnp.float32)


def _decode(u8, wdec):
    return pl.pallas_call(
        _dec_body,
        grid=(N_PAD // MB,),
        in_specs=[
            pl.BlockSpec((MB, HID_CH), lambda i: (i, 0)),
            pl.BlockSpec((OUT_CH, HID_CH), lambda i: (0, 0)),
        ],
        out_specs=pl.BlockSpec((MB, OUT_CH), lambda i: (i, 0)),
        out_shape=jax.ShapeDtypeStruct((N_PAD, OUT_CH), jnp.float32),
    )(u8, wdec)


_mesh = plsc.VectorSubcoreMesh(core_axis_name="c", subcore_axis_name="s",
                               num_cores=NC, num_subcores=NS)


def _make_prop(n_iter):
  @functools.partial(
      pl.kernel,
      out_type=jax.ShapeDtypeStruct((4 * N_PAD, CBLK), jnp.float32),
      mesh=_mesh,
      compiler_params=pltpu.CompilerParams(use_tc_tiling_on_sc=False),
      scratch_types=[
          pltpu.VMEM((ROWS_PT, CBLK), jnp.float32),
          pltpu.VMEM((K,), jnp.int32),
          pltpu.VMEM((K,), jnp.int32),
          pltpu.VMEM((K,), jnp.float32),
          pltpu.VMEM((K, CBLK), jnp.float32),
          pltpu.VMEM((RCH, CBLK), jnp.float32),
          pltpu.VMEM((RCH, CBLK), jnp.float32),
          pltpu.VMEM((NS, 32), jnp.int32),
          pltpu.VMEM((2, 16), jnp.float32),
          pltpu.SemaphoreType.DMA,
      ],
  )
  def _prop_sc(src_hbm, dst_hbm, w_hbm, binj_hbm, starts_hbm, pars_hbm,
               out_hbm, agg_v, src_v, dst_v, w_v, rows_v, ub_v, bb_v,
               starts_v, pars_v, sem):
    c = lax.axis_index("c")
    s = lax.axis_index("s")
    pltpu.sync_copy(pars_hbm, pars_v)
    pltpu.sync_copy(starts_hbm, starts_v)
    beta = pars_v[0, :]
    gamma = pars_v[1, :]
    omb = jnp.ones((16,), jnp.float32) - beta
    zero16 = jnp.zeros((16,), jnp.float32)
    iota16 = lax.iota(jnp.int32, 16)

    r0 = s * ROWS_PT
    s_lo = starts_v[s, pl.ds(0, 16)][0]
    s_hi = starts_v[s, pl.ds(16, 16)][0]
    abase = (s_lo // K) * K
    nch = (s_hi - abase + K - 1) // K

    for blk in range(NBLK_PER_CORE):
        cb = c * NBLK_PER_CORE + blk
        ub0 = cb * N_PAD

        def _init(k, cr):
            rb = r0 + k * RCH
            pltpu.sync_copy(binj_hbm.at[pl.ds(ub0 + rb, RCH)], bb_v)

            def _row(r, cu):
                for q in range(CBLK // 16):
                    sl = pl.ds(q * 16, 16)
                    ub_v[r, sl] = beta * jnp.maximum(bb_v[r, sl], 0.0)
                return cu
            lax.fori_loop(0, RCH, _row, 0)
            pltpu.sync_copy(ub_v, out_hbm.at[pl.ds(ub0 + rb, RCH)])
            return cr
        lax.fori_loop(0, NRCH, _init, 0)
        plsc.subcore_barrier()

        def _iter(it, carry):
            def _zrow(r, cz):
                for q in range(CBLK // 16):
                    agg_v[r, pl.ds(q * 16, 16)] = zero16
                return cz
            lax.fori_loop(0, ROWS_PT, _zrow, 0)

            def _chunk(ci, cc):
                e0 = abase + ci * K
                pltpu.sync_copy(src_hbm.at[pl.ds(e0, K)], src_v)
                pltpu.sync_copy(dst_hbm.at[pl.ds(e0, K)], dst_v)
                pltpu.sync_copy(w_hbm.at[pl.ds(e0, K)], w_v)

                def _prep(i, cp):
                    sl = pl.ds(i * 16, 16)
                    eidx = jnp.full((16,), e0 + i * 16, jnp.int32) + iota16
                    live = (eidx >= s_lo) & (eidx < s_hi)
                    w_v[sl] = jnp.where(live, w_v[sl] * gamma, 0.0)
                    src_v[sl] = src_v[sl] + ub0
                    dl = dst_v[sl] - r0
                    dst_v[sl] = jnp.clip(dl, 0, ROWS_PT - 1)
                    return cp
                lax.fori_loop(0, K // 16, _prep, 0)

                pltpu.async_copy(out_hbm.at[src_v], rows_v, sem).wait()

                def _acc(gi, cm):
                    wrow = w_v[pl.ds(gi * 16, 16)]
                    drow = dst_v[pl.ds(gi * 16, 16)]
                    for e16 in range(16):
                        r = gi * 16 + e16
                        wv = jnp.full((16,), wrow[e16], jnp.float32)
                        dr = drow[e16]
                        for q in range(CBLK // 16):
                            sl = pl.ds(q * 16, 16)
                            agg_v[dr, sl] = agg_v[dr, sl] + rows_v[r, sl] * wv
                    return cm
                lax.fori_loop(0, K // 16, _acc, 0)
                return cc
            lax.fori_loop(0, nch, _chunk, 0)
            plsc.subcore_barrier()

            def _updk(k, cr):
                rb = r0 + k * RCH
                pltpu.sync_copy(binj_hbm.at[pl.ds(ub0 + rb, RCH)], bb_v)
                pltpu.sync_copy(out_hbm.at[pl.ds(ub0 + rb, RCH)], ub_v)
                kr = k * RCH

                def _row(r, cu):
                    for q in range(CBLK // 16):
                        sl = pl.ds(q * 16, 16)
                        un = omb * ub_v[r, sl] + beta * jnp.maximum(
                            agg_v[kr + r, sl] + bb_v[r, sl], 0.0)
                        ub_v[r, sl] = un
                    return cu
                lax.fori_loop(0, RCH, _row, 0)
                pltpu.sync_copy(ub_v, out_hbm.at[pl.ds(ub0 + rb, RCH)])
                return cr
            lax.fori_loop(0, NRCH, _updk, 0)
            plsc.subcore_barrier()
            return carry
        lax.fori_loop(1, n_iter, _iter, 0)

  return _prop_sc


_prop_full = _make_prop(MAX_ITER)


def kernel(x, edge_index, edge_weight, Wenc, benc, Wb, Wdec, beta_raw, gamma_raw):
    x = x.astype(jnp.float32)
    src = edge_index[0].astype(jnp.int32)
    dst = edge_index[1].astype(jnp.int32)
    w = edge_weight.astype(jnp.float32)

    order = jnp.argsort(dst)
    srcs = jnp.pad(src[order], (0, EPAD - N_EDGES))
    dsts = jnp.pad(dst[order], (0, EPAD - N_EDGES))
    ws = jnp.pad(w[order], (0, EPAD - N_EDGES))
    bounds = jnp.arange(NS + 1, dtype=jnp.int32) * ROWS_PT
    starts = jnp.searchsorted(dsts[:N_EDGES], bounds, side="left").astype(jnp.int32)
    starts = starts.at[NS].set(N_EDGES)
    starts_exp = jnp.concatenate(
        [jnp.repeat(starts[:NS, None], 16, axis=1),
         jnp.repeat(starts[1:NS + 1, None], 16, axis=1)], axis=1)

    xp = jnp.pad(x, ((0, N_PAD - N_NODES), (0, 0)))
    beta = jax.nn.sigmoid(beta_raw.astype(jnp.float32))
    gamma = jax.nn.sigmoid(gamma_raw.astype(jnp.float32))
    pars = jnp.stack([jnp.full((16,), beta), jnp.full((16,), gamma)])

    binj = _compute_binj(xp, Wenc, benc.reshape(1, HID_CH), Wb)
    binj4 = binj.reshape(N_PAD, 4, CBLK).transpose(1, 0, 2).reshape(4 * N_PAD, CBLK)
    u84 = _prop_full(srcs, dsts, ws, binj4, starts_exp, pars)
    u8 = u84.reshape(4, N_PAD, CBLK).transpose(1, 0, 2).reshape(N_PAD, HID_CH)
    out = _decode(u8, Wdec)
    return out[:N_NODES]


# speedup vs baseline: 1.9663x; 1.4928x over previous
import functools

import jax
import jax.numpy as jnp
from jax import lax
from jax.experimental import pallas as pl
from jax.experimental.pallas import tpu as pltpu
from jax.experimental.pallas import tpu_sc as plsc

N_NODES = 10000
N_PAD = 10240
N_EDGES = 320000
IN_CH = 128
HID_CH = 256
OUT_CH = 128
MAX_ITER = 8

NC = 2
NS = 16
CBLK = 64
NBLK_PER_CORE = 2

K = 128
EPAD = N_EDGES + K
ROWS_PT = N_PAD // NS
RCH = 64
NRCH = ROWS_PT // RCH

MB = 1024


def _binj_body(x_ref, wenc_ref, benc_ref, wb_ref, o_ref):
    h = lax.dot_general(x_ref[...], wenc_ref[...], (((1,), (1,)), ((), ())),
                        preferred_element_type=jnp.float32)
    h = h + benc_ref[...]
    o_ref[...] = lax.dot_general(h, wb_ref[...], (((1,), (1,)), ((), ())),
                                 preferred_element_type=jnp.float32)


def _compute_binj(xp, wenc, benc2, wb):
    return pl.pallas_call(
        _binj_body,
        grid=(N_PAD // MB,),
        in_specs=[
            pl.BlockSpec((MB, IN_CH), lambda i: (i, 0)),
            pl.BlockSpec((HID_CH, IN_CH), lambda i: (0, 0)),
            pl.BlockSpec((1, HID_CH), lambda i: (0, 0)),
            pl.BlockSpec((HID_CH, HID_CH), lambda i: (0, 0)),
        ],
        out_specs=pl.BlockSpec((MB, HID_CH), lambda i: (i, 0)),
        out_shape=jax.ShapeDtypeStruct((N_PAD, HID_CH), jnp.float32),
    )(xp, wenc, benc2, wb)


def _dec_body(u_ref, wd_ref, o_ref):
    o_ref[...] = lax.dot_general(jnp.maximum(u_ref[...], 0.0), wd_ref[...],
                                 (((1,), (1,)), ((), ())),
                                 preferred_element_type=jnp.float32)


def _decode(u8, wdec):
    return pl.pallas_call(
        _dec_body,
        grid=(N_PAD // MB,),
        in_specs=[
            pl.BlockSpec((MB, HID_CH), lambda i: (i, 0)),
            pl.BlockSpec((OUT_CH, HID_CH), lambda i: (0, 0)),
        ],
        out_specs=pl.BlockSpec((MB, OUT_CH), lambda i: (i, 0)),
        out_shape=jax.ShapeDtypeStruct((N_PAD, OUT_CH), jnp.float32),
    )(u8, wdec)


_mesh = plsc.VectorSubcoreMesh(core_axis_name="c", subcore_axis_name="s",
                               num_cores=NC, num_subcores=NS)


def _make_prop(n_iter):
  @functools.partial(
      pl.kernel,
      out_type=jax.ShapeDtypeStruct((4 * N_PAD, CBLK), jnp.float32),
      mesh=_mesh,
      compiler_params=pltpu.CompilerParams(use_tc_tiling_on_sc=False),
      scratch_types=[
          pltpu.VMEM((ROWS_PT, CBLK), jnp.float32),
          pltpu.VMEM((2, 2, K), jnp.int32),
          pltpu.VMEM((2, K), jnp.float32),
          pltpu.VMEM((2, K, CBLK), jnp.float32),
          pltpu.VMEM((RCH, CBLK), jnp.float32),
          pltpu.VMEM((RCH, CBLK), jnp.float32),
          pltpu.VMEM((NS, 32), jnp.int32),
          pltpu.VMEM((2, 16), jnp.float32),
          pltpu.SemaphoreType.DMA,
          pltpu.SemaphoreType.DMA,
      ],
  )
  def _prop_sc(ed_hbm, w_hbm, binj_hbm, starts_hbm, pars_hbm,
               out_hbm, agg_v, ed_v, w2_v, rows_v, ub_v, bb_v,
               starts_v, pars_v, sem0, sem1):
    c = lax.axis_index("c")
    s = lax.axis_index("s")
    pltpu.sync_copy(pars_hbm, pars_v)
    pltpu.sync_copy(starts_hbm, starts_v)
    beta = pars_v[0, :]
    gamma = pars_v[1, :]
    omb = jnp.ones((16,), jnp.float32) - beta
    zero16 = jnp.zeros((16,), jnp.float32)
    iota16 = lax.iota(jnp.int32, 16)

    r0 = s * ROWS_PT
    s_lo = starts_v[s, pl.ds(0, 16)][0]
    s_hi = starts_v[s, pl.ds(16, 16)][0]
    abase = (s_lo // K) * K
    nch = (s_hi - abase + K - 1) // K

    for blk in range(NBLK_PER_CORE):
        cb = c * NBLK_PER_CORE + blk
        ub0 = cb * N_PAD

        def _init(k, cr):
            rb = r0 + k * RCH
            pltpu.sync_copy(binj_hbm.at[pl.ds(ub0 + rb, RCH)], bb_v)

            def _row(r, cu):
                for q in range(CBLK // 16):
                    sl = pl.ds(q * 16, 16)
                    ub_v[r, sl] = beta * jnp.maximum(bb_v[r, sl], 0.0)
                return cu
            lax.fori_loop(0, RCH, _row, 0)
            pltpu.sync_copy(ub_v, out_hbm.at[pl.ds(ub0 + rb, RCH)])
            return cr
        lax.fori_loop(0, NRCH, _init, 0)
        plsc.subcore_barrier()

        def _iter(it, carry):
            def _zrow(r, cz):
                for q in range(CBLK // 16):
                    agg_v[r, pl.ds(q * 16, 16)] = zero16
                return cz
            lax.fori_loop(0, ROWS_PT, _zrow, 0)

            def _stage(ci, b):
                e0 = abase + ci * K
                sem = sem0 if b == 0 else sem1
                pltpu.sync_copy(ed_hbm.at[:, pl.ds(e0, K)], ed_v.at[b])
                pltpu.sync_copy(w_hbm.at[pl.ds(e0, K)], w2_v.at[b])

                def _prep(i, cp):
                    sl = pl.ds(i * 16, 16)
                    eidx = jnp.full((16,), e0 + i * 16, jnp.int32) + iota16
                    live = (eidx >= s_lo) & (eidx < s_hi)
                    w2_v[b, sl] = jnp.where(live, w2_v[b, sl] * gamma, 0.0)
                    ed_v[b, 0, sl] = ed_v[b, 0, sl] + ub0
                    ed_v[b, 1, sl] = jnp.clip(ed_v[b, 1, sl] - r0, 0, ROWS_PT - 1)
                    return cp
                lax.fori_loop(0, K // 16, _prep, 0)
                pltpu.async_copy(out_hbm.at[ed_v.at[b, 0]], rows_v.at[b], sem)

            def _acc(b):
                sem = sem0 if b == 0 else sem1
                pltpu.make_async_copy(out_hbm.at[ed_v.at[b, 0]],
                                      rows_v.at[b], sem).wait()

                def _accg(gi, cm):
                    gsl = pl.ds(gi * 16, 16)
                    wrow = w2_v[b, gsl]
                    drow = ed_v[b, 1, gsl]
                    for e16 in range(16):
                        r = gi * 16 + e16
                        wv = jnp.full((16,), wrow[e16], jnp.float32)
                        dr = drow[e16]
                        for q in range(CBLK // 16):
                            sl = pl.ds(q * 16, 16)
                            plsc.addupdate(agg_v.at[dr, sl],
                                           rows_v[b, r, sl] * wv)
                    return cm
                lax.fori_loop(0, K // 16, _accg, 0)

            @pl.when(nch > 0)
            def _prologue():
                _stage(0, 0)

            def _pair(cp, cc):
                for b in range(2):
                    ci = cp * 2 + b

                    @pl.when(ci < nch)
                    def _do():
                        @pl.when(ci + 1 < nch)
                        def _next():
                            _stage(ci + 1, 1 - b)
                        _acc(b)
                return cc
            lax.fori_loop(0, (nch + 1) // 2, _pair, 0)
            plsc.subcore_barrier()

            def _updk(k, cr):
                rb = r0 + k * RCH
                pltpu.sync_copy(binj_hbm.at[pl.ds(ub0 + rb, RCH)], bb_v)
                pltpu.sync_copy(out_hbm.at[pl.ds(ub0 + rb, RCH)], ub_v)
                kr = k * RCH

                def _row(r, cu):
                    for q in range(CBLK // 16):
                        sl = pl.ds(q * 16, 16)
                        un = omb * ub_v[r, sl] + beta * jnp.maximum(
                            agg_v[kr + r, sl] + bb_v[r, sl], 0.0)
                        ub_v[r, sl] = un
                    return cu
                lax.fori_loop(0, RCH, _row, 0)
                pltpu.sync_copy(ub_v, out_hbm.at[pl.ds(ub0 + rb, RCH)])
                return cr
            lax.fori_loop(0, NRCH, _updk, 0)
            plsc.subcore_barrier()
            return carry
        lax.fori_loop(1, n_iter, _iter, 0)

  return _prop_sc


_prop_full = _make_prop(MAX_ITER)


def kernel(x, edge_index, edge_weight, Wenc, benc, Wb, Wdec, beta_raw, gamma_raw):
    x = x.astype(jnp.float32)
    src = edge_index[0].astype(jnp.int32)
    dst = edge_index[1].astype(jnp.int32)
    w = edge_weight.astype(jnp.float32)

    order = jnp.argsort(dst)
    srcs = jnp.pad(src[order], (0, EPAD - N_EDGES))
    dsts = jnp.pad(dst[order], (0, EPAD - N_EDGES))
    ws = jnp.pad(w[order], (0, EPAD - N_EDGES))
    edata = jnp.stack([srcs, dsts])
    bounds = jnp.arange(NS + 1, dtype=jnp.int32) * ROWS_PT
    starts = jnp.searchsorted(dsts[:N_EDGES], bounds, side="left").astype(jnp.int32)
    starts = starts.at[NS].set(N_EDGES)
    starts_exp = jnp.concatenate(
        [jnp.repeat(starts[:NS, None], 16, axis=1),
         jnp.repeat(starts[1:NS + 1, None], 16, axis=1)], axis=1)

    xp = jnp.pad(x, ((0, N_PAD - N_NODES), (0, 0)))
    beta = jax.nn.sigmoid(beta_raw.astype(jnp.float32))
    gamma = jax.nn.sigmoid(gamma_raw.astype(jnp.float32))
    pars = jnp.stack([jnp.full((16,), beta), jnp.full((16,), gamma)])

    binj = _compute_binj(xp, Wenc, benc.reshape(1, HID_CH), Wb)
    binj4 = binj.reshape(N_PAD, 4, CBLK).transpose(1, 0, 2).reshape(4 * N_PAD, CBLK)
    u84 = _prop_full(edata, ws, binj4, starts_exp, pars)
    u8 = u84.reshape(4, N_PAD, CBLK).transpose(1, 0, 2).reshape(N_PAD, HID_CH)
    out = _decode(u8, Wdec)
    return out[:N_NODES]


# speedup vs baseline: 2.1006x; 1.0683x over previous
import functools

import jax
import jax.numpy as jnp
from jax import lax
from jax.experimental import pallas as pl
from jax.experimental.pallas import tpu as pltpu
from jax.experimental.pallas import tpu_sc as plsc

N_NODES = 10000
N_PAD = 10240
N_EDGES = 320000
IN_CH = 128
HID_CH = 256
OUT_CH = 128
MAX_ITER = 8

NC = 2
NS = 16
CBLK = 64
NBLK_PER_CORE = 2

K = 128
EPAD = N_EDGES + K
ROWS_PT = N_PAD // NS
RCH = 64
NRCH = ROWS_PT // RCH

MB = 1024


def _binj_body(x_ref, wenc_ref, benc_ref, wb_ref, o_ref):
    h = lax.dot_general(x_ref[...], wenc_ref[...], (((1,), (1,)), ((), ())),
                        preferred_element_type=jnp.float32)
    h = h + benc_ref[...]
    o_ref[...] = lax.dot_general(h, wb_ref[...], (((1,), (1,)), ((), ())),
                                 preferred_element_type=jnp.float32)


def _compute_binj(xp, wenc, benc2, wb):
    return pl.pallas_call(
        _binj_body,
        grid=(N_PAD // MB,),
        in_specs=[
            pl.BlockSpec((MB, IN_CH), lambda i: (i, 0)),
            pl.BlockSpec((HID_CH, IN_CH), lambda i: (0, 0)),
            pl.BlockSpec((1, HID_CH), lambda i: (0, 0)),
            pl.BlockSpec((HID_CH, HID_CH), lambda i: (0, 0)),
        ],
        out_specs=pl.BlockSpec((MB, HID_CH), lambda i: (i, 0)),
        out_shape=jax.ShapeDtypeStruct((N_PAD, HID_CH), jnp.float32),
    )(xp, wenc, benc2, wb)


def _dec_body(u_ref, wd_ref, o_ref):
    o_ref[...] = lax.dot_general(jnp.maximum(u_ref[...], 0.0), wd_ref[...],
                                 (((1,), (1,)), ((), ())),
                                 preferred_element_type=jnp.float32)


def _decode(u8, wdec):
    return pl.pallas_call(
        _dec_body,
        grid=(N_PAD // MB,),
        in_specs=[
            pl.BlockSpec((MB, HID_CH), lambda i: (i, 0)),
            pl.BlockSpec((OUT_CH, HID_CH), lambda i: (0, 0)),
        ],
        out_specs=pl.BlockSpec((MB, OUT_CH), lambda i: (i, 0)),
        out_shape=jax.ShapeDtypeStruct((N_PAD, OUT_CH), jnp.float32),
    )(u8, wdec)


_mesh = plsc.VectorSubcoreMesh(core_axis_name="c", subcore_axis_name="s",
                               num_cores=NC, num_subcores=NS)


def _make_prop(n_iter):
  @functools.partial(
      pl.kernel,
      out_type=jax.ShapeDtypeStruct((4 * N_PAD, CBLK), jnp.float32),
      mesh=_mesh,
      compiler_params=pltpu.CompilerParams(use_tc_tiling_on_sc=False),
      scratch_types=[
          pltpu.VMEM((ROWS_PT, CBLK), jnp.float32),
          pltpu.VMEM((2, 2, K), jnp.int32),
          pltpu.VMEM((2, K), jnp.float32),
          pltpu.VMEM((2, K, CBLK), jnp.float32),
          pltpu.VMEM((RCH, CBLK), jnp.float32),
          pltpu.VMEM((RCH, CBLK), jnp.float32),
          pltpu.VMEM((NS, 32), jnp.int32),
          pltpu.VMEM((2, 16), jnp.float32),
          pltpu.SemaphoreType.DMA,
          pltpu.SemaphoreType.DMA,
      ],
  )
  def _prop_sc(ed_hbm, w_hbm, binj_hbm, starts_hbm, pars_hbm,
               out_hbm, agg_v, ed_v, w2_v, rows_v, ub_v, bb_v,
               starts_v, pars_v, sem0, sem1):
    c = lax.axis_index("c")
    s = lax.axis_index("s")
    pltpu.sync_copy(pars_hbm, pars_v)
    pltpu.sync_copy(starts_hbm, starts_v)
    beta = pars_v[0, :]
    gamma = pars_v[1, :]
    omb = jnp.ones((16,), jnp.float32) - beta
    zero16 = jnp.zeros((16,), jnp.float32)
    iota16 = lax.iota(jnp.int32, 16)

    r0 = s * ROWS_PT
    s_lo = starts_v[s, pl.ds(0, 16)][0]
    s_hi = starts_v[s, pl.ds(16, 16)][0]
    abase = (s_lo // K) * K
    nch = (s_hi - abase + K - 1) // K

    for blk in range(NBLK_PER_CORE):
        cb = c * NBLK_PER_CORE + blk
        ub0 = cb * N_PAD

        def _init(k, cr):
            rb = r0 + k * RCH
            pltpu.sync_copy(binj_hbm.at[pl.ds(ub0 + rb, RCH)], bb_v)

            def _row(r, cu):
                for q in range(CBLK // 16):
                    sl = pl.ds(q * 16, 16)
                    ub_v[r, sl] = beta * jnp.maximum(bb_v[r, sl], 0.0)
                return cu
            lax.fori_loop(0, RCH, _row, 0)
            pltpu.sync_copy(ub_v, out_hbm.at[pl.ds(ub0 + rb, RCH)])
            return cr
        lax.fori_loop(0, NRCH, _init, 0)
        plsc.subcore_barrier()

        def _iter(it, carry):
            def _zrow(r, cz):
                for q in range(CBLK // 16):
                    agg_v[r, pl.ds(q * 16, 16)] = zero16
                return cz
            lax.fori_loop(0, ROWS_PT, _zrow, 0)

            def _stage(ci, b):
                e0 = abase + ci * K
                sem = sem0 if b == 0 else sem1
                pltpu.sync_copy(ed_hbm.at[:, pl.ds(e0, K)], ed_v.at[b])
                pltpu.sync_copy(w_hbm.at[pl.ds(e0, K)], w2_v.at[b])

                def _prep(i, cp):
                    sl = pl.ds(i * 16, 16)
                    eidx = jnp.full((16,), e0 + i * 16, jnp.int32) + iota16
                    live = (eidx >= s_lo) & (eidx < s_hi)
                    w2_v[b, sl] = jnp.where(live, w2_v[b, sl] * gamma, 0.0)
                    ed_v[b, 0, sl] = ed_v[b, 0, sl] + ub0
                    ed_v[b, 1, sl] = jnp.clip(ed_v[b, 1, sl] - r0, 0, ROWS_PT - 1)
                    return cp
                lax.fori_loop(0, K // 16, _prep, 0)
                pltpu.async_copy(out_hbm.at[ed_v.at[b, 0]], rows_v.at[b], sem)

            def _acc(b):
                sem = sem0 if b == 0 else sem1
                pltpu.make_async_copy(out_hbm.at[ed_v.at[b, 0]],
                                      rows_v.at[b], sem).wait()

                @plsc.parallel_loop(0, K // 16, unroll=2)
                def _accg(gi):
                    gsl = pl.ds(gi * 16, 16)
                    wrow = w2_v[b, gsl]
                    drow = ed_v[b, 1, gsl]
                    for e16 in range(16):
                        r = gi * 16 + e16
                        wv = jnp.full((16,), wrow[e16], jnp.float32)
                        dr = drow[e16]
                        for q in range(CBLK // 16):
                            sl = pl.ds(q * 16, 16)
                            plsc.addupdate(agg_v.at[dr, sl],
                                           rows_v[b, r, sl] * wv)

            @pl.when(nch > 0)
            def _prologue():
                _stage(0, 0)

            def _pair(cp, cc):
                for b in range(2):
                    ci = cp * 2 + b

                    @pl.when(ci < nch)
                    def _do():
                        @pl.when(ci + 1 < nch)
                        def _next():
                            _stage(ci + 1, 1 - b)
                        _acc(b)
                return cc
            lax.fori_loop(0, (nch + 1) // 2, _pair, 0)
            plsc.subcore_barrier()

            def _updk(k, cr):
                rb = r0 + k * RCH
                pltpu.sync_copy(binj_hbm.at[pl.ds(ub0 + rb, RCH)], bb_v)
                pltpu.sync_copy(out_hbm.at[pl.ds(ub0 + rb, RCH)], ub_v)
                kr = k * RCH

                def _row(r, cu):
                    for q in range(CBLK // 16):
                        sl = pl.ds(q * 16, 16)
                        un = omb * ub_v[r, sl] + beta * jnp.maximum(
                            agg_v[kr + r, sl] + bb_v[r, sl], 0.0)
                        ub_v[r, sl] = un
                    return cu
                lax.fori_loop(0, RCH, _row, 0)
                pltpu.sync_copy(ub_v, out_hbm.at[pl.ds(ub0 + rb, RCH)])
                return cr
            lax.fori_loop(0, NRCH, _updk, 0)
            plsc.subcore_barrier()
            return carry
        lax.fori_loop(1, n_iter, _iter, 0)

  return _prop_sc


_prop_full = _make_prop(MAX_ITER)


def kernel(x, edge_index, edge_weight, Wenc, benc, Wb, Wdec, beta_raw, gamma_raw):
    x = x.astype(jnp.float32)
    src = edge_index[0].astype(jnp.int32)
    dst = edge_index[1].astype(jnp.int32)
    w = edge_weight.astype(jnp.float32)

    order = jnp.argsort(dst)
    srcs = jnp.pad(src[order], (0, EPAD - N_EDGES))
    dsts = jnp.pad(dst[order], (0, EPAD - N_EDGES))
    ws = jnp.pad(w[order], (0, EPAD - N_EDGES))
    edata = jnp.stack([srcs, dsts])
    bounds = jnp.arange(NS + 1, dtype=jnp.int32) * ROWS_PT
    starts = jnp.searchsorted(dsts[:N_EDGES], bounds, side="left").astype(jnp.int32)
    starts = starts.at[NS].set(N_EDGES)
    starts_exp = jnp.concatenate(
        [jnp.repeat(starts[:NS, None], 16, axis=1),
         jnp.repeat(starts[1:NS + 1, None], 16, axis=1)], axis=1)

    xp = jnp.pad(x, ((0, N_PAD - N_NODES), (0, 0)))
    beta = jax.nn.sigmoid(beta_raw.astype(jnp.float32))
    gamma = jax.nn.sigmoid(gamma_raw.astype(jnp.float32))
    pars = jnp.stack([jnp.full((16,), beta), jnp.full((16,), gamma)])

    binj = _compute_binj(xp, Wenc, benc.reshape(1, HID_CH), Wb)
    binj4 = binj.reshape(N_PAD, 4, CBLK).transpose(1, 0, 2).reshape(4 * N_PAD, CBLK)
    u84 = _prop_full(edata, ws, binj4, starts_exp, pars)
    u8 = u84.reshape(4, N_PAD, CBLK).transpose(1, 0, 2).reshape(N_PAD, HID_CH)
    out = _decode(u8, Wdec)
    return out[:N_NODES]


# speedup vs baseline: 2.3891x; 1.1374x over previous
import functools

import jax
import jax.numpy as jnp
from jax import lax
from jax.experimental import pallas as pl
from jax.experimental.pallas import tpu as pltpu
from jax.experimental.pallas import tpu_sc as plsc

N_NODES = 10000
N_PAD = 10240
N_EDGES = 320000
IN_CH = 128
HID_CH = 256
OUT_CH = 128
MAX_ITER = 8

NC = 2
NS = 16
CBLK = 64
NBLK_PER_CORE = 2

K = 128
EPAD = N_EDGES + K
ROWS_PT = N_PAD // NS
RCH = 64
NRCH = ROWS_PT // RCH

MB = 1024


def _binj_body(x_ref, wenc_ref, benc_ref, wb_ref, o_ref):
    h = lax.dot_general(x_ref[...], wenc_ref[...], (((1,), (1,)), ((), ())),
                        preferred_element_type=jnp.float32)
    h = h + benc_ref[...]
    o_ref[...] = lax.dot_general(h, wb_ref[...], (((1,), (1,)), ((), ())),
                                 preferred_element_type=jnp.float32)


def _compute_binj(xp, wenc, benc2, wb):
    return pl.pallas_call(
        _binj_body,
        grid=(N_PAD // MB,),
        in_specs=[
            pl.BlockSpec((MB, IN_CH), lambda i: (i, 0)),
            pl.BlockSpec((HID_CH, IN_CH), lambda i: (0, 0)),
            pl.BlockSpec((1, HID_CH), lambda i: (0, 0)),
            pl.BlockSpec((HID_CH, HID_CH), lambda i: (0, 0)),
        ],
        out_specs=pl.BlockSpec((MB, HID_CH), lambda i: (i, 0)),
        out_shape=jax.ShapeDtypeStruct((N_PAD, HID_CH), jnp.float32),
    )(xp, wenc, benc2, wb)


def _dec_body(u_ref, wd_ref, o_ref):
    o_ref[...] = lax.dot_general(jnp.maximum(u_ref[...], 0.0), wd_ref[...],
                                 (((1,), (1,)), ((), ())),
                                 preferred_element_type=jnp.float32)


def _decode(u8, wdec):
    return pl.pallas_call(
        _dec_body,
        grid=(N_PAD // MB,),
        in_specs=[
            pl.BlockSpec((MB, HID_CH), lambda i: (i, 0)),
            pl.BlockSpec((OUT_CH, HID_CH), lambda i: (0, 0)),
        ],
        out_specs=pl.BlockSpec((MB, OUT_CH), lambda i: (i, 0)),
        out_shape=jax.ShapeDtypeStruct((N_PAD, OUT_CH), jnp.float32),
    )(u8, wdec)


_mesh = plsc.VectorSubcoreMesh(core_axis_name="c", subcore_axis_name="s",
                               num_cores=NC, num_subcores=NS)


def _make_prop(n_iter):
  @functools.partial(
      pl.kernel,
      out_type=jax.ShapeDtypeStruct((4 * N_PAD, CBLK), jnp.float32),
      mesh=_mesh,
      compiler_params=pltpu.CompilerParams(use_tc_tiling_on_sc=False),
      scratch_types=[
          pltpu.VMEM((ROWS_PT, CBLK), jnp.float32),
          pltpu.VMEM((2, 2, K), jnp.int32),
          pltpu.VMEM((2, K), jnp.float32),
          pltpu.VMEM((2, K, CBLK), jnp.float32),
          pltpu.VMEM((RCH, CBLK), jnp.float32),
          pltpu.VMEM((RCH, CBLK), jnp.float32),
          pltpu.VMEM((NS, 32), jnp.int32),
          pltpu.VMEM((2, 16), jnp.float32),
          pltpu.SemaphoreType.DMA,
          pltpu.SemaphoreType.DMA,
      ],
  )
  def _prop_sc(ed_hbm, w_hbm, binj_hbm, starts_hbm, pars_hbm,
               out_hbm, agg_v, ed_v, w2_v, rows_v, ub_v, bb_v,
               starts_v, pars_v, sem0, sem1):
    c = lax.axis_index("c")
    s = lax.axis_index("s")
    pltpu.sync_copy(pars_hbm, pars_v)
    pltpu.sync_copy(starts_hbm, starts_v)
    beta = pars_v[0, :]
    gamma = pars_v[1, :]
    omb = jnp.ones((16,), jnp.float32) - beta
    zero16 = jnp.zeros((16,), jnp.float32)
    iota16 = lax.iota(jnp.int32, 16)

    r0 = s * ROWS_PT
    s_lo = starts_v[s, pl.ds(0, 16)][0]
    s_hi = starts_v[s, pl.ds(16, 16)][0]
    abase = (s_lo // K) * K
    nch = (s_hi - abase + K - 1) // K

    for blk in range(NBLK_PER_CORE):
        cb = c * NBLK_PER_CORE + blk
        ub0 = cb * N_PAD

        def _init(k, cr):
            rb = r0 + k * RCH
            pltpu.sync_copy(binj_hbm.at[pl.ds(ub0 + rb, RCH)], bb_v)

            def _row(r, cu):
                for q in range(CBLK // 16):
                    sl = pl.ds(q * 16, 16)
                    ub_v[r, sl] = beta * jnp.maximum(bb_v[r, sl], 0.0)
                return cu
            lax.fori_loop(0, RCH, _row, 0)
            pltpu.sync_copy(ub_v, out_hbm.at[pl.ds(ub0 + rb, RCH)])
            return cr
        lax.fori_loop(0, NRCH, _init, 0)
        plsc.subcore_barrier()

        def _iter(it, carry):
            def _zrow(r, cz):
                for q in range(CBLK // 16):
                    agg_v[r, pl.ds(q * 16, 16)] = zero16
                return cz
            lax.fori_loop(0, ROWS_PT, _zrow, 0)

            def _stage(ci, b):
                e0 = abase + ci * K
                sem = sem0 if b == 0 else sem1
                pltpu.sync_copy(ed_hbm.at[:, pl.ds(e0, K)], ed_v.at[b])
                pltpu.sync_copy(w_hbm.at[pl.ds(e0, K)], w2_v.at[b])

                def _prep(i, cp):
                    sl = pl.ds(i * 16, 16)
                    eidx = jnp.full((16,), e0 + i * 16, jnp.int32) + iota16
                    live = (eidx >= s_lo) & (eidx < s_hi)
                    w2_v[b, sl] = jnp.where(live, w2_v[b, sl] * gamma, 0.0)
                    ed_v[b, 0, sl] = ed_v[b, 0, sl] + ub0
                    ed_v[b, 1, sl] = jnp.clip(ed_v[b, 1, sl] - r0, 0, ROWS_PT - 1)
                    return cp
                lax.fori_loop(0, K // 16, _prep, 0)
                pltpu.async_copy(out_hbm.at[ed_v.at[b, 0]], rows_v.at[b], sem)

            def _acc(b):
                sem = sem0 if b == 0 else sem1
                pltpu.make_async_copy(out_hbm.at[ed_v.at[b, 0]],
                                      rows_v.at[b], sem).wait()

                @plsc.parallel_loop(0, K // 16, unroll=4)
                def _accg(gi):
                    gsl = pl.ds(gi * 16, 16)
                    wrow = w2_v[b, gsl]
                    drow = ed_v[b, 1, gsl]
                    for e16 in range(16):
                        r = gi * 16 + e16
                        wv = jnp.full((16,), wrow[e16], jnp.float32)
                        dr = drow[e16]
                        for q in range(CBLK // 16):
                            sl = pl.ds(q * 16, 16)
                            plsc.addupdate(agg_v.at[dr, sl],
                                           rows_v[b, r, sl] * wv)

            @pl.when(nch > 0)
            def _prologue():
                _stage(0, 0)

            def _pair(cp, cc):
                for b in range(2):
                    ci = cp * 2 + b

                    @pl.when(ci < nch)
                    def _do():
                        @pl.when(ci + 1 < nch)
                        def _next():
                            _stage(ci + 1, 1 - b)
                        _acc(b)
                return cc
            lax.fori_loop(0, (nch + 1) // 2, _pair, 0)
            plsc.subcore_barrier()

            def _updk(k, cr):
                rb = r0 + k * RCH
                pltpu.sync_copy(binj_hbm.at[pl.ds(ub0 + rb, RCH)], bb_v)
                pltpu.sync_copy(out_hbm.at[pl.ds(ub0 + rb, RCH)], ub_v)
                kr = k * RCH

                def _row(r, cu):
                    for q in range(CBLK // 16):
                        sl = pl.ds(q * 16, 16)
                        un = omb * ub_v[r, sl] + beta * jnp.maximum(
                            agg_v[kr + r, sl] + bb_v[r, sl], 0.0)
                        ub_v[r, sl] = un
                    return cu
                lax.fori_loop(0, RCH, _row, 0)
                pltpu.sync_copy(ub_v, out_hbm.at[pl.ds(ub0 + rb, RCH)])
                return cr
            lax.fori_loop(0, NRCH, _updk, 0)
            plsc.subcore_barrier()
            return carry
        lax.fori_loop(1, n_iter, _iter, 0)

  return _prop_sc


_prop_full = _make_prop(MAX_ITER)


def kernel(x, edge_index, edge_weight, Wenc, benc, Wb, Wdec, beta_raw, gamma_raw):
    x = x.astype(jnp.float32)
    src = edge_index[0].astype(jnp.int32)
    dst = edge_index[1].astype(jnp.int32)
    w = edge_weight.astype(jnp.float32)

    order = jnp.argsort(dst)
    srcs = jnp.pad(src[order], (0, EPAD - N_EDGES))
    dsts = jnp.pad(dst[order], (0, EPAD - N_EDGES))
    ws = jnp.pad(w[order], (0, EPAD - N_EDGES))
    edata = jnp.stack([srcs, dsts])
    bounds = jnp.arange(NS + 1, dtype=jnp.int32) * ROWS_PT
    starts = jnp.searchsorted(dsts[:N_EDGES], bounds, side="left").astype(jnp.int32)
    starts = starts.at[NS].set(N_EDGES)
    starts_exp = jnp.concatenate(
        [jnp.repeat(starts[:NS, None], 16, axis=1),
         jnp.repeat(starts[1:NS + 1, None], 16, axis=1)], axis=1)

    xp = jnp.pad(x, ((0, N_PAD - N_NODES), (0, 0)))
    beta = jax.nn.sigmoid(beta_raw.astype(jnp.float32))
    gamma = jax.nn.sigmoid(gamma_raw.astype(jnp.float32))
    pars = jnp.stack([jnp.full((16,), beta), jnp.full((16,), gamma)])

    binj = _compute_binj(xp, Wenc, benc.reshape(1, HID_CH), Wb)
    binj4 = binj.reshape(N_PAD, 4, CBLK).transpose(1, 0, 2).reshape(4 * N_PAD, CBLK)
    u84 = _prop_full(edata, ws, binj4, starts_exp, pars)
    u8 = u84.reshape(4, N_PAD, CBLK).transpose(1, 0, 2).reshape(N_PAD, HID_CH)
    out = _decode(u8, Wdec)
    return out[:N_NODES]
